# Initial kernel scaffold; baseline (speedup 1.0000x reference)
import jax
import jax.numpy as jnp
from jax.experimental import pallas as pl


def kernel(x, hyperedge_index, batch, W1, b1, W2, b2, g1, be1, g2, be2, Wf, bf):
    raise NotImplementedError("write your pallas kernel here")


# speedup vs baseline: 7.3750x; 7.3750x over previous
import functools

import jax
import jax.numpy as jnp
from jax import lax
from jax.experimental import pallas as pl
from jax.experimental.pallas import tpu as pltpu
from jax.experimental.pallas import tpu_sc as plsc

_NC = 2
_NS = 16
_LANES = 16
_K = 80


def _sc_segment_sum(values, gather_idx, scatter_idx, num_segments):
  nnz = gather_idx.shape[0]
  d = values.shape[1]
  nw = _NC * _NS
  per_w = nnz // nw
  n_chunks = per_w // _K
  seg_chunks = num_segments // _K
  mesh = plsc.VectorSubcoreMesh(core_axis_name="c", subcore_axis_name="s")

  @functools.partial(
      pl.kernel,
      out_type=jax.ShapeDtypeStruct((_NC, num_segments, d), jnp.float32),
      mesh=mesh,
      scratch_types=[
          pltpu.VMEM((_K,), jnp.int32),
          pltpu.VMEM((_K,), jnp.int32),
          pltpu.VMEM((_K, d), jnp.float32),
          pltpu.VMEM((_K, d), jnp.float32),
          pltpu.VMEM_SHARED((num_segments, d), jnp.float32),
      ],
  )
  def kern(vals_hbm, gidx_hbm, sidx_hbm, out_hbm, gi_v, si_v, rows_v, zb_v,
           acc_sh):
    c = lax.axis_index("c")
    s = lax.axis_index("s")
    wid = c * _NS + s

    @pl.loop(0, _K)
    def _zrow(r):
      @pl.loop(0, d, step=_LANES)
      def _zcol(col):
        zb_v[pl.ds(r, 1), pl.ds(col, _LANES)] = jnp.zeros(
            (1, _LANES), jnp.float32)

    @pl.loop(s, seg_chunks, step=_NS)
    def _zinit(jc):
      pltpu.sync_copy(zb_v, acc_sh.at[pl.ds(jc * _K, _K)])

    plsc.subcore_barrier()

    base = wid * per_w

    @pl.loop(0, n_chunks)
    def _edge_batch(j):
      off = base + j * _K
      pltpu.sync_copy(gidx_hbm.at[pl.ds(off, _K)], gi_v)
      pltpu.sync_copy(sidx_hbm.at[pl.ds(off, _K)], si_v)
      pltpu.sync_copy(vals_hbm.at[gi_v], rows_v)
      pltpu.sync_copy(rows_v, acc_sh.at[si_v], add=True)

    plsc.subcore_barrier()

    @pl.loop(s, seg_chunks, step=_NS)
    def _writeout(jc):
      r0 = jc * _K
      pltpu.sync_copy(acc_sh.at[pl.ds(r0, _K)], out_hbm.at[c].at[pl.ds(r0, _K)])

  return kern(values, gather_idx, scatter_idx)


def _sc_degrees(node_idx, edge_idx, n_nodes, n_edges):
  nnz = node_idx.shape[0]
  nw = _NC * _NS
  per_w = nnz // nw
  n_chunks = per_w // _K
  node_chunks = n_nodes // _K
  edge_chunks = n_edges // _K
  mesh = plsc.VectorSubcoreMesh(core_axis_name="c", subcore_axis_name="s")

  @functools.partial(
      pl.kernel,
      out_type=[
          jax.ShapeDtypeStruct((_NC, n_nodes, _LANES), jnp.float32),
          jax.ShapeDtypeStruct((_NC, n_edges, _LANES), jnp.float32),
      ],
      mesh=mesh,
      scratch_types=[
          pltpu.VMEM((_K,), jnp.int32),
          pltpu.VMEM((_K,), jnp.int32),
          pltpu.VMEM((_K, _LANES), jnp.float32),
          pltpu.VMEM((_K, _LANES), jnp.float32),
          pltpu.VMEM_SHARED((n_nodes, _LANES), jnp.float32),
          pltpu.VMEM_SHARED((n_edges, _LANES), jnp.float32),
      ],
  )
  def kern(nidx_hbm, eidx_hbm, outn_hbm, oute_hbm, ni_v, ei_v, ones_v, zb_v,
           hn_sh, he_sh):
    c = lax.axis_index("c")
    s = lax.axis_index("s")
    wid = c * _NS + s

    @pl.loop(0, _K)
    def _fill(r):
      ones_v[pl.ds(r, 1), pl.ds(0, _LANES)] = jnp.ones((1, _LANES),
                                                       jnp.float32)
      zb_v[pl.ds(r, 1), pl.ds(0, _LANES)] = jnp.zeros((1, _LANES),
                                                      jnp.float32)

    @pl.loop(s, node_chunks, step=_NS)
    def _zn(jc):
      pltpu.sync_copy(zb_v, hn_sh.at[pl.ds(jc * _K, _K)])

    @pl.loop(s, edge_chunks, step=_NS)
    def _ze(jc):
      pltpu.sync_copy(zb_v, he_sh.at[pl.ds(jc * _K, _K)])

    plsc.subcore_barrier()

    base = wid * per_w

    @pl.loop(0, n_chunks)
    def _edge_batch(j):
      off = base + j * _K
      pltpu.sync_copy(nidx_hbm.at[pl.ds(off, _K)], ni_v)
      pltpu.sync_copy(eidx_hbm.at[pl.ds(off, _K)], ei_v)
      pltpu.sync_copy(ones_v, hn_sh.at[ni_v], add=True)
      pltpu.sync_copy(ones_v, he_sh.at[ei_v], add=True)

    plsc.subcore_barrier()

    @pl.loop(s, node_chunks, step=_NS)
    def _wn(jc):
      r0 = jc * _K
      pltpu.sync_copy(hn_sh.at[pl.ds(r0, _K)], outn_hbm.at[c].at[pl.ds(r0, _K)])

    @pl.loop(s, edge_chunks, step=_NS)
    def _we(jc):
      r0 = jc * _K
      pltpu.sync_copy(he_sh.at[pl.ds(r0, _K)], oute_hbm.at[c].at[pl.ds(r0, _K)])

  return kern(node_idx, edge_idx)


def _tc_linear(x, w, b2d):
  def body(x_ref, w_ref, b_ref, o_ref):
    o_ref[...] = jnp.dot(x_ref[...], w_ref[...],
                         preferred_element_type=jnp.float32) + b_ref[...]

  return pl.pallas_call(
      body,
      out_shape=jax.ShapeDtypeStruct((x.shape[0], w.shape[1]), jnp.float32),
  )(x, w, b2d)


def _inv_deg(dp_ref):
  deg = dp_ref[0, :, 0:1] + dp_ref[1, :, 0:1]
  return jnp.where(deg > 0, 1.0 / deg, 0.0)


def _tc_combine_scale(parts, deg_parts):
  def body(p_ref, dp_ref, o_ref):
    o_ref[...] = (p_ref[0] + p_ref[1]) * _inv_deg(dp_ref)

  s, d = parts.shape[1], parts.shape[2]
  return pl.pallas_call(
      body,
      out_shape=jax.ShapeDtypeStruct((s, d), jnp.float32),
  )(parts, deg_parts)


def _tc_scale_bn_silu_linear(parts, deg_parts, g2d, be2d, w, b2d):
  def body(p_ref, dp_ref, g_ref, be_ref, w_ref, b_ref, o_ref):
    h = (p_ref[0] + p_ref[1]) * _inv_deg(dp_ref)
    mu = jnp.mean(h, axis=0, keepdims=True)
    var = jnp.mean((h - mu) * (h - mu), axis=0, keepdims=True)
    h = g_ref[...] * (h - mu) * lax.rsqrt(var + 1e-5) + be_ref[...]
    h = h * jax.nn.sigmoid(h)
    o_ref[...] = jnp.dot(h, w_ref[...],
                         preferred_element_type=jnp.float32) + b_ref[...]

  s = parts.shape[1]
  return pl.pallas_call(
      body,
      out_shape=jax.ShapeDtypeStruct((s, w.shape[1]), jnp.float32),
  )(parts, deg_parts, g2d, be2d, w, b2d)


def _tc_final(parts, deg_parts, g2d, be2d, batch2d, n_graphs, wf, bf2d, d):
  s = parts.shape[1]

  def body(p_ref, dp_ref, g_ref, be_ref, b_ref, wf_ref, bf_ref, o_ref):
    h = ((p_ref[0] + p_ref[1]) * _inv_deg(dp_ref))[:, :d]
    mu = jnp.mean(h, axis=0, keepdims=True)
    var = jnp.mean((h - mu) * (h - mu), axis=0, keepdims=True)
    h = g_ref[...] * (h - mu) * lax.rsqrt(var + 1e-5) + be_ref[...]

    batch = b_ref[...]
    gids = lax.broadcasted_iota(jnp.int32, (s, n_graphs), 1)
    onehot = (batch == gids).astype(jnp.float32)
    cnt = lax.dot_general(onehot, jnp.ones((s, 1), jnp.float32),
                          (((0,), (0,)), ((), ())),
                          preferred_element_type=jnp.float32)
    sums = lax.dot_general(onehot, h, (((0,), (0,)), ((), ())),
                           preferred_element_type=jnp.float32)
    mean = sums / jnp.maximum(cnt, 1.0)

    maxs = []
    for gi in range(n_graphs):
      m = jnp.where(batch == gi, h, -jnp.inf)
      maxs.append(jnp.max(m, axis=0, keepdims=True))
    mx = jnp.concatenate(maxs, axis=0)

    pooled = jnp.concatenate([mean, mx], axis=1)
    o_ref[...] = jnp.dot(pooled, wf_ref[...],
                         preferred_element_type=jnp.float32) + bf_ref[...]

  return pl.pallas_call(
      body,
      out_shape=jax.ShapeDtypeStruct((n_graphs, wf.shape[1]), jnp.float32),
  )(parts, deg_parts, g2d, be2d, batch2d, wf, bf2d)


def kernel(x, hyperedge_index, batch, W1, b1, W2, b2, g1, be1, g2, be2, Wf,
           bf):
  n_nodes = x.shape[0]
  node_idx = hyperedge_index[0].astype(jnp.int32)
  edge_idx = hyperedge_index[1].astype(jnp.int32)
  n_edges = n_nodes
  n_graphs = 16
  batch2d = batch.astype(jnp.int32).reshape(-1, 1)

  hid2 = W2.shape[1]
  pad = W1.shape[1] - hid2
  W2p = jnp.pad(W2, ((0, 0), (0, pad)))
  b2p = jnp.pad(b2, ((0, pad),))
  deg_n_p, deg_e_p = _sc_degrees(node_idx, edge_idx, n_nodes, n_edges)
  h0 = _tc_linear(x, W1, b1.reshape(1, -1))

  p = _sc_segment_sum(h0, node_idx, edge_idx, n_edges)
  e_feat = _tc_combine_scale(p, deg_e_p)
  p = _sc_segment_sum(e_feat, edge_idx, node_idx, n_nodes)
  h1 = _tc_scale_bn_silu_linear(p, deg_n_p, g1.reshape(1, -1),
                                be1.reshape(1, -1), W2p, b2p.reshape(1, -1))

  p = _sc_segment_sum(h1, node_idx, edge_idx, n_edges)
  e_feat = _tc_combine_scale(p, deg_e_p)
  p = _sc_segment_sum(e_feat, edge_idx, node_idx, n_nodes)

  return _tc_final(p, deg_n_p, g2.reshape(1, -1), be2.reshape(1, -1),
                   batch2d, n_graphs, Wf, bf.reshape(1, 1), hid2)
